# baseline (speedup 1.0000x reference)
import dataclasses
import functools

import jax
import jax.numpy as jnp
from jax import lax
from jax.experimental import pallas as pl
from jax.experimental.pallas import tpu as pltpu
from jax.experimental.pallas import tpu_sc as plsc

N = 10000
NP = 10240
NA = 5
E = 160000
EP = 163840
CH = 128
D0 = 128
DH = 512

_MESH = plsc.VectorSubcoreMesh(core_axis_name="c", subcore_axis_name="s")
f32 = jnp.float32
i32 = jnp.int32

_CP0 = pltpu.CompilerParams(use_tc_tiling_on_sc=False)
_CP = dataclasses.replace(_CP0, needs_layout_passes=False)


@functools.partial(
    pl.kernel,
    out_type=jax.ShapeDtypeStruct((2, NP, 16), f32),
    mesh=_MESH,
    compiler_params=_CP0,
    scratch_types=[
        pltpu.VMEM((40, CH), i32),
        pltpu.VMEM((NA, CH, 16), f32),
        pltpu.VMEM((CH, 16), f32),
        pltpu.VMEM_SHARED((NP, 16), f32),
    ],
)
def _deg_kernel(dstd, onesg, z16, out, idx_v, ones_v, zbuf, acc):
    c = lax.axis_index("c")
    s = lax.axis_index("s")
    w = s * 2 + c
    pltpu.sync_copy(onesg, ones_v)
    pltpu.sync_copy(z16, zbuf)
    rows_per_tile = NP // 16
    base = s * rows_per_tile

    @pl.loop(0, rows_per_tile // CH)
    def _(j):
        pltpu.sync_copy(zbuf, acc.at[pl.ds(base + j * CH, CH)])

    plsc.subcore_barrier()

    for g in range(NA):
        pltpu.sync_copy(dstd.at[g, w], idx_v)

        @pl.loop(0, 40)
        def _(k):
            pltpu.sync_copy(ones_v.at[g], acc.at[idx_v.at[k]],
                            add=True)

    plsc.subcore_barrier()
    pltpu.sync_copy(acc.at[pl.ds(base, rows_per_tile)],
                    out.at[c, pl.ds(base, rows_per_tile)])


@functools.partial(
    pl.kernel,
    out_type=jax.ShapeDtypeStruct((NA * 2 * NP, D0), f32),
    mesh=_MESH,
    compiler_params=_CP0,
    scratch_types=[
        pltpu.VMEM((40, CH), i32),
        pltpu.VMEM((40, CH), i32),
        pltpu.VMEM((CH, D0), f32),
        pltpu.VMEM((CH, D0), f32),
        pltpu.VMEM_SHARED((NP, D0), f32),
        pltpu.SemaphoreType.DMA,
        pltpu.SemaphoreType.DMA,
        pltpu.SemaphoreType.DMA,
        pltpu.SemaphoreType.DMA,
    ],
)
def _scatter_kernel(u2, src2, dst2, zrows, out, src_v, dst_v, rows0, rows1,
                    acc, g0, g1, s0, s1):
    c = lax.axis_index("c")
    s = lax.axis_index("s")
    rows_per_tile = NP // 16
    base = s * rows_per_tile

    def gath(k, buf, sem):
        pltpu.async_copy(u2.at[src_v.at[k]], buf, sem)

    def gwait(buf, sem):
        pltpu.make_async_copy(u2.at[src_v.at[0]], buf, sem).wait()

    def scat(k, buf, sem):
        pltpu.async_copy(buf, acc.at[dst_v.at[k]], sem, add=True)

    def swait(buf, sem):
        pltpu.make_async_copy(buf, acc.at[dst_v.at[0]], sem).wait()

    for g in range(NA):
        pltpu.sync_copy(zrows, rows0)

        @pl.loop(0, rows_per_tile // CH)
        def _(j):
            pltpu.sync_copy(rows0, acc.at[pl.ds(base + j * CH, CH)])

        plsc.subcore_barrier()

        for p in range(2):
            pltpu.sync_copy(src2.at[g, c, s, pl.ds(p * 40, 40)], src_v)
            pltpu.sync_copy(dst2.at[g, s, pl.ds(p * 40, 40)], dst_v)

            gath(0, rows0, g0)
            gwait(rows0, g0)
            gath(1, rows1, g1)
            scat(0, rows0, s0)

            @pl.loop(0, 19)
            def _(j):
                k = 1 + 2 * j
                swait(rows0, s0)
                gwait(rows1, g1)
                gath(k + 1, rows0, g0)
                scat(k, rows1, s1)
                swait(rows1, s1)
                gwait(rows0, g0)
                gath(k + 2, rows1, g1)
                scat(k + 1, rows0, s0)

            swait(rows0, s0)
            gwait(rows1, g1)
            scat(39, rows1, s1)
            swait(rows1, s1)

        plsc.subcore_barrier()
        obase = (g * 2 + c) * NP + base
        pltpu.sync_copy(acc.at[pl.ds(base, rows_per_tile)],
                        out.at[pl.ds(obase, rows_per_tile)])
        plsc.subcore_barrier()


@functools.partial(
    pl.kernel,
    out_type=jax.ShapeDtypeStruct((2, NP, 16), f32),
    mesh=_MESH,
    compiler_params=_CP,
    scratch_types=[
        pltpu.VMEM((NA * NP,), f32),
        pltpu.VMEM((40, CH), i32),
        pltpu.VMEM((40, CH), i32),
        pltpu.VMEM((CH, 16), f32),
        pltpu.VMEM_SHARED((NP, 16), f32),
    ],
)
def _t_scatter_kernel(t_tab, src3, dst3, z16, out, t_v, src_v, dst_v,
                      buf, acc):
    c = lax.axis_index("c")
    s = lax.axis_index("s")
    w = s * 2 + c
    pltpu.sync_copy(t_tab, t_v)
    pltpu.sync_copy(z16, buf)
    rows_per_tile = NP // 16
    base = s * rows_per_tile

    @pl.loop(0, rows_per_tile // CH)
    def _(j):
        pltpu.sync_copy(buf, acc.at[pl.ds(base + j * CH, CH)])

    plsc.subcore_barrier()
    lane = lax.iota(i32, 16)
    zero16 = jnp.zeros((16,), f32)

    for g in range(NA):
        colg = jnp.full((16,), g, i32)
        pltpu.sync_copy(src3.at[g, w], src_v)
        pltpu.sync_copy(dst3.at[g, w], dst_v)

        @pl.loop(0, CH)
        def _(r):
            buf[r, pl.ds(0, 16)] = zero16

        @pl.loop(0, 40)
        def _(k):
            @pl.loop(0, 8)
            def _(j):
                sidx = src_v[k, pl.ds(j * 16, 16)]
                vals = plsc.load_gather(t_v, [g * NP + sidx])
                plsc.store_scatter(buf, [j * 16 + lane, colg], vals)

            pltpu.sync_copy(buf, acc.at[dst_v.at[k]],
                            add=True)

    plsc.subcore_barrier()
    pltpu.sync_copy(acc.at[pl.ds(base, rows_per_tile)],
                    out.at[c, pl.ds(base, rows_per_tile)])


_BLK = 1024


def _dense_body(a0, a1, u0, u1, dv, w1, b1r, w2r, t16o):
    z0 = dv[...] * (a0[...] + u0[...])
    z1 = dv[...] * (a1[...] + u1[...])
    w = w1[...]
    h = lax.dot_general(z0, w[:D0], (((1,), (0,)), ((), ())),
                        precision=lax.Precision.HIGHEST)
    h = h + lax.dot_general(z1, w[D0:], (((1,), (0,)), ((), ())),
                            precision=lax.Precision.HIGHEST)
    h = h + b1r[...]
    r = jnp.maximum(h, 0.0)
    y = jnp.sum(r * w2r[...], axis=1, keepdims=True)
    t16o[...] = jnp.broadcast_to(dv[...] * y, (_BLK, 16))


def _dense_tc(acc_flat, u_flat, dinv_col, W1, b1, w2row):
    grid = (NA, NP // _BLK)
    return pl.pallas_call(
        _dense_body,
        grid=grid,
        in_specs=[
            pl.BlockSpec((_BLK, D0), lambda g, i: (g * 2 * (NP // _BLK) + i, 0)),
            pl.BlockSpec((_BLK, D0),
                         lambda g, i: ((g * 2 + 1) * (NP // _BLK) + i, 0)),
            pl.BlockSpec((_BLK, D0), lambda g, i: (g * 2 * (NP // _BLK) + i, 0)),
            pl.BlockSpec((_BLK, D0),
                         lambda g, i: ((g * 2 + 1) * (NP // _BLK) + i, 0)),
            pl.BlockSpec((_BLK, 1), lambda g, i: (g * (NP // _BLK) + i, 0)),
            pl.BlockSpec((2 * D0, DH), lambda g, i: (0, 0)),
            pl.BlockSpec((1, DH), lambda g, i: (0, 0)),
            pl.BlockSpec((1, DH), lambda g, i: (0, 0)),
        ],
        out_specs=pl.BlockSpec((_BLK, 16), lambda g, i: (g * (NP // _BLK) + i, 0)),
        out_shape=jax.ShapeDtypeStruct((NA * NP, 16), f32),
    )(acc_flat, acc_flat, u_flat, u_flat, dinv_col, W1, b1, w2row)


def kernel(x, edge_index, W1, b1, W2, b2):
    ei = edge_index.astype(i32)
    pad = jnp.full((NA, 2, EP - E), N, dtype=i32)
    eip = jnp.concatenate([ei, pad], axis=2)

    z16 = jnp.zeros((CH, 16), f32)
    zrows = jnp.zeros((CH, D0), f32)
    onesg = (jnp.arange(16)[None, :] == jnp.arange(NA)[:, None]).astype(f32)
    onesg = jnp.broadcast_to(onesg[:, None, :], (NA, CH, 16))

    src_t = eip[:, 0].reshape(NA, 32, 40, CH)
    dst_t = eip[:, 1].reshape(NA, 32, 40, CH)

    deg_parts = _deg_kernel(dst_t, onesg, z16)
    deg = deg_parts[0, :N, :NA] + deg_parts[1, :N, :NA] + 1.0
    dinv = (1.0 / jnp.sqrt(deg)).T

    u = dinv[:, :, None] * x
    u_halves = jnp.stack([u[:, :, :D0], u[:, :, D0:]], axis=1)
    u2 = jnp.pad(u_halves, ((0, 0), (0, 0), (0, NP - N), (0, 0)))
    u2 = u2.reshape(NA * 2 * NP, D0)

    offs = (jnp.arange(NA * 2, dtype=i32) * NP).reshape(NA, 2, 1)
    src2 = (eip[:, 0][:, None, :] + offs).reshape(NA, 2, 16, 80, CH)
    dst2 = eip[:, 1].reshape(NA, 16, 80, CH)
    acc_flat = _scatter_kernel(u2, src2, dst2, zrows)

    b1r = b1.reshape(1, DH)
    w2row = W2.reshape(1, DH)
    dv_col = jnp.pad(dinv, ((0, 0), (0, NP - N))).reshape(NA * NP, 1)
    t16 = _dense_tc(acc_flat, u2, dv_col, W1, b1r, w2row)
    t_tab = t16[:, 0].reshape(NA, NP)

    s_parts = _t_scatter_kernel(t_tab.reshape(-1), src_t, dst_t, z16)
    S = s_parts[0, :N, :NA] + s_parts[1, :N, :NA]

    scores = []
    for i in range(NA):
        scores.append(
            jnp.sum(dinv[i] * (S[:, i] + t_tab[i, :N])) / N + b2[0])
    return jax.nn.log_softmax(jnp.stack(scores), axis=0)


# speedup vs baseline: 1.0512x; 1.0512x over previous
import dataclasses
import functools

import jax
import jax.numpy as jnp
from jax import lax
from jax.experimental import pallas as pl
from jax.experimental.pallas import tpu as pltpu
from jax.experimental.pallas import tpu_sc as plsc

N = 10000
NP = 10240
NA = 5
E = 160000
EP = 163840
CH = 128
D0 = 128
DH = 512

_MESH = plsc.VectorSubcoreMesh(core_axis_name="c", subcore_axis_name="s")
f32 = jnp.float32
i32 = jnp.int32

_CP0 = pltpu.CompilerParams(use_tc_tiling_on_sc=False)
_CP = dataclasses.replace(_CP0, needs_layout_passes=False)


@functools.partial(
    pl.kernel,
    out_type=jax.ShapeDtypeStruct((2, NP, 16), f32),
    mesh=_MESH,
    compiler_params=_CP0,
    scratch_types=[
        pltpu.VMEM((40, CH), i32),
        pltpu.VMEM((NA, CH, 16), f32),
        pltpu.VMEM((CH, 16), f32),
        pltpu.VMEM_SHARED((NP, 16), f32),
    ],
)
def _deg_kernel(dstd, onesg, z16, out, idx_v, ones_v, zbuf, acc):
    c = lax.axis_index("c")
    s = lax.axis_index("s")
    w = s * 2 + c
    pltpu.sync_copy(onesg, ones_v)
    pltpu.sync_copy(z16, zbuf)
    rows_per_tile = NP // 16
    base = s * rows_per_tile

    @pl.loop(0, rows_per_tile // CH)
    def _(j):
        pltpu.sync_copy(zbuf, acc.at[pl.ds(base + j * CH, CH)])

    plsc.subcore_barrier()

    for g in range(NA):
        pltpu.sync_copy(dstd.at[w, pl.ds(g * 40, 40)], idx_v)

        @pl.loop(0, 40)
        def _(k):
            pltpu.sync_copy(ones_v.at[g], acc.at[idx_v.at[k]],
                            add=True)

    plsc.subcore_barrier()
    pltpu.sync_copy(acc.at[pl.ds(base, rows_per_tile)],
                    out.at[c, pl.ds(base, rows_per_tile)])


@functools.partial(
    pl.kernel,
    out_type=jax.ShapeDtypeStruct((NA * 2 * NP, D0), f32),
    mesh=_MESH,
    compiler_params=_CP0,
    scratch_types=[
        pltpu.VMEM((40, CH), i32),
        pltpu.VMEM((40, CH), i32),
        pltpu.VMEM((CH, D0), f32),
        pltpu.VMEM((CH, D0), f32),
        pltpu.VMEM_SHARED((NP, D0), f32),
        pltpu.SemaphoreType.DMA,
        pltpu.SemaphoreType.DMA,
        pltpu.SemaphoreType.DMA,
        pltpu.SemaphoreType.DMA,
    ],
)
def _scatter_kernel(u2, src2, dst2, zrows, out, src_v, dst_v, rows0, rows1,
                    acc, g0, g1, s0, s1):
    c = lax.axis_index("c")
    s = lax.axis_index("s")
    rows_per_tile = NP // 16
    base = s * rows_per_tile

    def gath(k, buf, sem):
        pltpu.async_copy(u2.at[src_v.at[k]], buf, sem)

    def gwait(buf, sem):
        pltpu.make_async_copy(u2.at[src_v.at[0]], buf, sem).wait()

    def scat(k, buf, sem):
        pltpu.async_copy(buf, acc.at[dst_v.at[k]], sem, add=True)

    def swait(buf, sem):
        pltpu.make_async_copy(buf, acc.at[dst_v.at[0]], sem).wait()

    for g in range(NA):
        pltpu.sync_copy(zrows, rows0)

        @pl.loop(0, rows_per_tile // CH)
        def _(j):
            pltpu.sync_copy(rows0, acc.at[pl.ds(base + j * CH, CH)])

        plsc.subcore_barrier()

        for p in range(2):
            pltpu.sync_copy(src2.at[g, c, s, pl.ds(p * 40, 40)], src_v)
            pltpu.sync_copy(dst2.at[g, s, pl.ds(p * 40, 40)], dst_v)

            gath(0, rows0, g0)
            gwait(rows0, g0)
            gath(1, rows1, g1)
            scat(0, rows0, s0)

            @pl.loop(0, 19)
            def _(j):
                k = 1 + 2 * j
                swait(rows0, s0)
                gwait(rows1, g1)
                gath(k + 1, rows0, g0)
                scat(k, rows1, s1)
                swait(rows1, s1)
                gwait(rows0, g0)
                gath(k + 2, rows1, g1)
                scat(k + 1, rows0, s0)

            swait(rows0, s0)
            gwait(rows1, g1)
            scat(39, rows1, s1)
            swait(rows1, s1)

        plsc.subcore_barrier()
        obase = (g * 2 + c) * NP + base
        pltpu.sync_copy(acc.at[pl.ds(base, rows_per_tile)],
                        out.at[pl.ds(obase, rows_per_tile)])
        plsc.subcore_barrier()


@functools.partial(
    pl.kernel,
    out_type=jax.ShapeDtypeStruct((2, NP, 16), f32),
    mesh=_MESH,
    compiler_params=_CP,
    scratch_types=[
        pltpu.VMEM((NA * NP,), f32),
        pltpu.VMEM((40, CH), i32),
        pltpu.VMEM((40, CH), i32),
        pltpu.VMEM((CH, 16), f32),
        pltpu.VMEM_SHARED((NP, 16), f32),
    ],
)
def _t_scatter_kernel(t_tab, src3, dst3, z16, out, t_v, src_v, dst_v,
                      buf, acc):
    c = lax.axis_index("c")
    s = lax.axis_index("s")
    w = s * 2 + c
    pltpu.sync_copy(t_tab, t_v)
    pltpu.sync_copy(z16, buf)
    rows_per_tile = NP // 16
    base = s * rows_per_tile

    @pl.loop(0, rows_per_tile // CH)
    def _(j):
        pltpu.sync_copy(buf, acc.at[pl.ds(base + j * CH, CH)])

    plsc.subcore_barrier()
    lane = lax.iota(i32, 16)
    zero16 = jnp.zeros((16,), f32)

    for g in range(NA):
        colg = jnp.full((16,), g, i32)
        pltpu.sync_copy(src3.at[w, pl.ds(g * 40, 40)], src_v)
        pltpu.sync_copy(dst3.at[w, pl.ds(g * 40, 40)], dst_v)

        @pl.loop(0, CH)
        def _(r):
            buf[r, pl.ds(0, 16)] = zero16

        @pl.loop(0, 40)
        def _(k):
            @pl.loop(0, 8)
            def _(j):
                sidx = src_v[k, pl.ds(j * 16, 16)]
                vals = plsc.load_gather(t_v, [g * NP + sidx])
                plsc.store_scatter(buf, [j * 16 + lane, colg], vals)

            pltpu.sync_copy(buf, acc.at[dst_v.at[k]],
                            add=True)

    plsc.subcore_barrier()
    pltpu.sync_copy(acc.at[pl.ds(base, rows_per_tile)],
                    out.at[c, pl.ds(base, rows_per_tile)])


_BLK = 1024


def _dense_body(a0, a1, u0, u1, dv, w1, b1r, w2r, t16o):
    z0 = dv[...] * (a0[...] + u0[...])
    z1 = dv[...] * (a1[...] + u1[...])
    w = w1[...]
    h = lax.dot_general(z0, w[:D0], (((1,), (0,)), ((), ())),
                        precision=lax.Precision.HIGHEST)
    h = h + lax.dot_general(z1, w[D0:], (((1,), (0,)), ((), ())),
                            precision=lax.Precision.HIGHEST)
    h = h + b1r[...]
    r = jnp.maximum(h, 0.0)
    y = jnp.sum(r * w2r[...], axis=1, keepdims=True)
    t16o[...] = jnp.broadcast_to(dv[...] * y, (_BLK, 16))


def _dense_tc(acc_flat, u_flat, dinv_col, W1, b1, w2row):
    grid = (NA, NP // _BLK)
    return pl.pallas_call(
        _dense_body,
        grid=grid,
        in_specs=[
            pl.BlockSpec((_BLK, D0), lambda g, i: (g * 2 * (NP // _BLK) + i, 0)),
            pl.BlockSpec((_BLK, D0),
                         lambda g, i: ((g * 2 + 1) * (NP // _BLK) + i, 0)),
            pl.BlockSpec((_BLK, D0), lambda g, i: (g * 2 * (NP // _BLK) + i, 0)),
            pl.BlockSpec((_BLK, D0),
                         lambda g, i: ((g * 2 + 1) * (NP // _BLK) + i, 0)),
            pl.BlockSpec((_BLK, 1), lambda g, i: (g * (NP // _BLK) + i, 0)),
            pl.BlockSpec((2 * D0, DH), lambda g, i: (0, 0)),
            pl.BlockSpec((1, DH), lambda g, i: (0, 0)),
            pl.BlockSpec((1, DH), lambda g, i: (0, 0)),
        ],
        out_specs=pl.BlockSpec((_BLK, 16), lambda g, i: (g * (NP // _BLK) + i, 0)),
        out_shape=jax.ShapeDtypeStruct((NA * NP, 16), f32),
    )(acc_flat, acc_flat, u_flat, u_flat, dinv_col, W1, b1, w2row)


def kernel(x, edge_index, W1, b1, W2, b2):
    ei = edge_index.astype(i32)
    pad = jnp.full((NA, 2, EP - E), N, dtype=i32)
    eip = jnp.concatenate([ei, pad], axis=2)

    z16 = jnp.zeros((CH, 16), f32)
    zrows = jnp.zeros((CH, D0), f32)
    onesg = (jnp.arange(16)[None, :] == jnp.arange(NA)[:, None]).astype(f32)
    onesg = jnp.broadcast_to(onesg[:, None, :], (NA, CH, 16))

    src_t = jnp.transpose(eip[:, 0].reshape(NA, 32, 40, CH),
                          (1, 0, 2, 3)).reshape(32, NA * 40, CH)
    dst_t = jnp.transpose(eip[:, 1].reshape(NA, 32, 40, CH),
                          (1, 0, 2, 3)).reshape(32, NA * 40, CH)

    deg_parts = _deg_kernel(dst_t, onesg, z16)
    deg = deg_parts[0, :N, :NA] + deg_parts[1, :N, :NA] + 1.0
    dinv = (1.0 / jnp.sqrt(deg)).T

    u = dinv[:, :, None] * x
    u_halves = jnp.stack([u[:, :, :D0], u[:, :, D0:]], axis=1)
    u2 = jnp.pad(u_halves, ((0, 0), (0, 0), (0, NP - N), (0, 0)))
    u2 = u2.reshape(NA * 2 * NP, D0)

    offs = (jnp.arange(NA * 2, dtype=i32) * NP).reshape(NA, 2, 1)
    src2 = (eip[:, 0][:, None, :] + offs).reshape(NA, 2, 16, 80, CH)
    dst2 = eip[:, 1].reshape(NA, 16, 80, CH)
    acc_flat = _scatter_kernel(u2, src2, dst2, zrows)

    b1r = b1.reshape(1, DH)
    w2row = W2.reshape(1, DH)
    dv_col = jnp.pad(dinv, ((0, 0), (0, NP - N))).reshape(NA * NP, 1)
    t16 = _dense_tc(acc_flat, u2, dv_col, W1, b1r, w2row)
    t_tab = t16[:, 0].reshape(NA, NP)

    s_parts = _t_scatter_kernel(t_tab.reshape(-1), src_t, dst_t, z16)
    S = s_parts[0, :N, :NA] + s_parts[1, :N, :NA]

    scores = []
    for i in range(NA):
        scores.append(
            jnp.sum(dinv[i] * (S[:, i] + t_tab[i, :N])) / N + b2[0])
    return jax.nn.log_softmax(jnp.stack(scores), axis=0)
